# Initial kernel scaffold; baseline (speedup 1.0000x reference)
import jax
import jax.numpy as jnp
from jax.experimental import pallas as pl


def kernel(x_feature, edge_index, edge_weight, W0, b0, W1, b1, W2, b2):
    raise NotImplementedError("write your pallas kernel here")


# speedup vs baseline: 4.1243x; 4.1243x over previous
import functools

import jax
import jax.numpy as jnp
from jax import lax
from jax.experimental import pallas as pl
from jax.experimental.pallas import tpu as pltpu
from jax.experimental.pallas import tpu_sc as plsc

N = 10000
H = 128
E = 320000

NC = 2
NS = 16
NW = NC * NS

K = 128
R = (E + NW * K - 1) // (NW * K)
EPT = R * K
EPAD = NW * EPT
NPAD = 10240
ROWS_PER_TILE = NPAD // NS


def _spmm_body(sup_hbm, srcm_hbm, dstm_hbm, wm_hbm, out_hbm,
               src_v, dst_v, w_v, rows_v, acc_sh, sem):
  c = lax.axis_index("c")
  s = lax.axis_index("s")
  wid = s * NC + c

  pltpu.sync_copy(srcm_hbm.at[wid], src_v)
  pltpu.sync_copy(dstm_hbm.at[wid], dst_v)
  pltpu.sync_copy(wm_hbm.at[wid], w_v)

  def zrow(r, carry):
    for h in range(8):
      rows_v[r, pl.ds(h * 16, 16)] = jnp.zeros((16,), jnp.float32)
    return carry
  lax.fori_loop(0, K, zrow, 0)
  for i in range(ROWS_PER_TILE // K):
    pltpu.sync_copy(rows_v,
                    acc_sh.at[pl.ds(s * ROWS_PER_TILE + i * K, K)])
  plsc.subcore_barrier()

  def round_body(j, carry):
    pltpu.async_copy(sup_hbm.at[src_v.at[j]], rows_v, sem).wait()

    def scale_grp(g, carry2):
      wv16 = w_v[j, pl.ds(g * 16, 16)]
      base = g * 16
      for l in range(16):
        wv = jnp.full((16,), wv16[l])
        r = base + l
        for h in range(8):
          rows_v[r, pl.ds(h * 16, 16)] = rows_v[r, pl.ds(h * 16, 16)] * wv
      return carry2
    lax.fori_loop(0, K // 16, scale_grp, 0)

    pltpu.sync_copy(rows_v, acc_sh.at[dst_v.at[j]], add=True)
    return carry
  lax.fori_loop(0, R, round_body, 0)

  plsc.subcore_barrier()
  pltpu.sync_copy(acc_sh.at[pl.ds(s * ROWS_PER_TILE, ROWS_PER_TILE)],
                  out_hbm.at[c, pl.ds(s * ROWS_PER_TILE, ROWS_PER_TILE)])


def _sc_spmm(support, src2d, dst2d, w2d):
  mesh = plsc.VectorSubcoreMesh(core_axis_name="c", subcore_axis_name="s",
                                num_cores=NC, num_subcores=NS)
  fn = pl.kernel(
      _spmm_body,
      out_type=jax.ShapeDtypeStruct((NC, NPAD, H), jnp.float32),
      mesh=mesh,
      scratch_types=[
          pltpu.VMEM((R, K), jnp.int32),
          pltpu.VMEM((R, K), jnp.int32),
          pltpu.VMEM((R, K), jnp.float32),
          pltpu.VMEM((K, H), jnp.float32),
          pltpu.VMEM_SHARED((NPAD, H), jnp.float32),
          pltpu.SemaphoreType.DMA,
      ],
  )
  return fn(support, src2d, dst2d, w2d)


def _mm_body(x_ref, w_ref, o_ref):
  o_ref[...] = jnp.dot(x_ref[...], w_ref[...],
                       preferred_element_type=jnp.float32)


def _pair_norm(a):
  a = a - jnp.mean(a, axis=0, keepdims=True)
  rn = jnp.sqrt(1e-6 + jnp.sum(a * a, axis=1, keepdims=True))
  return a / rn


def _mid_body(p_ref, b_ref, w_ref, o_ref):
  a = p_ref[0, :N] + p_ref[1, :N] + b_ref[...]
  a = jnp.maximum(a, 0.0)
  a = _pair_norm(a)
  o_ref[...] = jnp.dot(a, w_ref[...], preferred_element_type=jnp.float32)


def _mid2_body(p_ref, b_ref, o_ref):
  a = p_ref[0, :N] + p_ref[1, :N] + b_ref[...]
  a = jnp.maximum(a, 0.0)
  o_ref[...] = _pair_norm(a)


def _final_body(p_ref, w_ref, b_ref, o_ref):
  a = p_ref[0, :N] + p_ref[1, :N]
  a = jnp.dot(a, w_ref[...], preferred_element_type=jnp.float32) + b_ref[...]
  a = _pair_norm(a)
  o_ref[...] = jax.nn.sigmoid(a)


def kernel(x_feature, edge_index, edge_weight, W0, b0, W1, b1, W2, b2):
  dst = edge_index[0]
  src = edge_index[1]
  pad = EPAD - E
  src2d = jnp.concatenate([src, jnp.zeros((pad,), jnp.int32)]).reshape(
      NW, R, K)
  dst2d = jnp.concatenate([dst, jnp.zeros((pad,), jnp.int32)]).reshape(
      NW, R, K)
  w2d = jnp.concatenate(
      [edge_weight, jnp.zeros((pad,), jnp.float32)]).reshape(NW, R, K)

  b0r = b0.reshape(1, H)
  b1r = b1.reshape(1, H)
  b2r = b2.reshape(1, -1)

  t = pl.pallas_call(
      _mm_body,
      out_shape=jax.ShapeDtypeStruct((N, H), jnp.float32),
  )(x_feature, W0)

  p = _sc_spmm(t, src2d, dst2d, w2d)

  t = pl.pallas_call(
      _mid_body,
      out_shape=jax.ShapeDtypeStruct((N, H), jnp.float32),
  )(p, b0r, W1)

  p = _sc_spmm(t, src2d, dst2d, w2d)

  t = pl.pallas_call(
      _mid2_body,
      out_shape=jax.ShapeDtypeStruct((N, H), jnp.float32),
  )(p, b1r)

  p = _sc_spmm(t, src2d, dst2d, w2d)

  out = pl.pallas_call(
      _final_body,
      out_shape=jax.ShapeDtypeStruct((N, W2.shape[1]), jnp.float32),
  )(p, W2, b2r)

  return out
